# baseline (speedup 1.0000x reference)
import functools

import jax
import jax.numpy as jnp
from jax import lax
from jax.experimental import pallas as pl
from jax.experimental.pallas import tpu as pltpu
from jax.experimental.pallas import tpu_sc as plsc

N_ABF = 8
N_DISC = 64
DOM_HI = 5.0
LOG2E = 1.4426950408889634


def _sc_gather(table, idx):
    n_rows = idx.shape[0]
    width = table.shape[1]
    nc, ns = 2, 16
    nw = nc * ns
    per_w = n_rows // nw

    mesh = plsc.VectorSubcoreMesh(core_axis_name="c", subcore_axis_name="s")

    @functools.partial(
        pl.kernel,
        mesh=mesh,
        out_type=jax.ShapeDtypeStruct((n_rows, width), jnp.float32),
        scratch_types=[
            pltpu.VMEM((per_w,), jnp.int32),
            pltpu.VMEM((per_w, width), jnp.float32),
            pltpu.SemaphoreType.DMA,
        ],
    )
    def gather_k(table_hbm, idx_hbm, out_hbm, idx_v, rows_v, sem):
        wid = lax.axis_index("s") * nc + lax.axis_index("c")
        base = wid * per_w
        pltpu.sync_copy(idx_hbm.at[pl.ds(base, per_w)], idx_v)
        pltpu.async_copy(table_hbm.at[idx_v], rows_v, sem).wait()
        pltpu.sync_copy(rows_v, out_hbm.at[pl.ds(base, per_w)])

    return gather_k(table, idx)


def _lane_consts(n):
    lane = lax.broadcasted_iota(jnp.int32, (1, 1, 2 * N_DISC), 2)
    sel = lane < N_DISC
    step = DOM_HI / (N_DISC - 1)
    dval = jnp.where(sel, lane, lane - N_DISC).astype(jnp.float32) * step
    return sel, dval


def _tc_main_body(r_ref, g_ref, o_ref):
    ni = r_ref.shape[1]
    n = r_ref.shape[2]
    sel, dval = _lane_consts(n)
    rb = r_ref[...]
    r0 = rb[0][:, :, None]
    r1 = rb[1][:, :, None]
    rp = jnp.where(sel, r0, r1)
    diff = rp - dval
    d2 = diff * diff
    g = g_ref[...]
    acc = jnp.zeros((ni, n, 2 * N_DISC), jnp.float32)
    for k in range(N_ABF):
        ce = g[0:n, k : k + 1].reshape(1, n, 1)
        co = g[n : 2 * n, k : k + 1].reshape(1, n, 1)
        ae = g[0:n, N_ABF + k : N_ABF + k + 1].reshape(1, n, 1)
        ao = g[n : 2 * n, N_ABF + k : N_ABF + k + 1].reshape(1, n, 1)
        cc = jnp.where(sel, ce, co)
        aa = jnp.where(sel, jnp.abs(ae), jnp.abs(ao)) * (-LOG2E)
        acc = acc + cc * jax.lax.exp2(aa * d2)
    o_ref[0] = acc[:, :, 0:N_DISC]
    o_ref[1] = acc[:, :, N_DISC : 2 * N_DISC]


def kernel(r, z, coeff_table, exp_table):
    b, n = z.shape
    v = coeff_table.shape[0]
    table = jnp.zeros((v, 128), jnp.float32)
    table = table.at[:, :N_ABF].set(coeff_table.astype(jnp.float32))
    table = table.at[:, N_ABF : 2 * N_ABF].set(exp_table.astype(jnp.float32))
    idx = z.astype(jnp.int32).reshape(-1)

    gathered = _sc_gather(table, idx)

    rs = r.reshape(b, n, n)
    ni = n // 2
    out = pl.pallas_call(
        _tc_main_body,
        grid=(b // 2, n // ni),
        in_specs=[
            pl.BlockSpec((2, ni, n), lambda i, q: (i, q, 0)),
            pl.BlockSpec((2 * n, 128), lambda i, q: (i, 0)),
        ],
        out_specs=pl.BlockSpec((2, ni, n, N_DISC), lambda i, q: (i, q, 0, 0)),
        out_shape=jax.ShapeDtypeStruct((b, n, n, N_DISC), jnp.float32),
    )(rs, gathered)
    return out


# speedup vs baseline: 1.0082x; 1.0082x over previous
import functools

import jax
import jax.numpy as jnp
from jax import lax
from jax.experimental import pallas as pl
from jax.experimental.pallas import tpu as pltpu
from jax.experimental.pallas import tpu_sc as plsc

N_ABF = 8
N_DISC = 64
DOM_HI = 5.0
LOG2E = 1.4426950408889634


def _sc_gather(table, idx):
    n_rows = idx.shape[0]
    width = table.shape[1]
    nc, ns = 2, 16
    nw = nc * ns
    per_w = n_rows // nw

    mesh = plsc.VectorSubcoreMesh(core_axis_name="c", subcore_axis_name="s")

    @functools.partial(
        pl.kernel,
        mesh=mesh,
        out_type=jax.ShapeDtypeStruct((n_rows, width), jnp.float32),
        scratch_types=[
            pltpu.VMEM((per_w,), jnp.int32),
            pltpu.VMEM((per_w, width), jnp.float32),
            pltpu.SemaphoreType.DMA,
        ],
    )
    def gather_k(table_hbm, idx_hbm, out_hbm, idx_v, rows_v, sem):
        wid = lax.axis_index("s") * nc + lax.axis_index("c")
        base = wid * per_w
        pltpu.sync_copy(idx_hbm.at[pl.ds(base, per_w)], idx_v)
        pltpu.async_copy(table_hbm.at[idx_v], rows_v, sem).wait()
        pltpu.sync_copy(rows_v, out_hbm.at[pl.ds(base, per_w)])

    return gather_k(table, idx)


def _lane_consts(n):
    lane = lax.broadcasted_iota(jnp.int32, (1, 1, 2 * N_DISC), 2)
    sel = lane < N_DISC
    step = DOM_HI / (N_DISC - 1)
    dval = jnp.where(sel, lane, lane - N_DISC).astype(jnp.float32) * step
    return sel, dval


def _tc_main_body(r_ref, g_ref, o_ref):
    ni = r_ref.shape[1]
    n = r_ref.shape[2]
    sel, dval = _lane_consts(n)
    rb = r_ref[...]
    r0 = rb[0][:, :, None]
    r1 = rb[1][:, :, None]
    rp = jnp.where(sel, r0, r1)
    diff = rp - dval
    d2 = diff * diff
    g = g_ref[...]
    acc = jnp.zeros((ni, n, 2 * N_DISC), jnp.float32)
    for k in range(N_ABF):
        ce = g[0:n, k : k + 1].reshape(1, n, 1)
        co = g[n : 2 * n, k : k + 1].reshape(1, n, 1)
        ae = g[0:n, N_ABF + k : N_ABF + k + 1].reshape(1, n, 1)
        ao = g[n : 2 * n, N_ABF + k : N_ABF + k + 1].reshape(1, n, 1)
        cc = jnp.where(sel, ce, co)
        aa = jnp.where(sel, jnp.abs(ae), jnp.abs(ao)) * (-LOG2E)
        acc = acc + cc * jax.lax.exp2(aa * d2)
    o_ref[0] = acc[:, :, 0:N_DISC]
    o_ref[1] = acc[:, :, N_DISC : 2 * N_DISC]


def kernel(r, z, coeff_table, exp_table):
    b, n = z.shape
    v = coeff_table.shape[0]
    table = jnp.zeros((v, 128), jnp.float32)
    table = table.at[:, :N_ABF].set(coeff_table.astype(jnp.float32))
    table = table.at[:, N_ABF : 2 * N_ABF].set(exp_table.astype(jnp.float32))
    idx = z.astype(jnp.int32).reshape(-1)

    gathered = _sc_gather(table, idx)

    rs = r.reshape(b, n, n)
    ni = n
    out = pl.pallas_call(
        _tc_main_body,
        grid=(b // 2, n // ni),
        in_specs=[
            pl.BlockSpec((2, ni, n), lambda i, q: (i, q, 0)),
            pl.BlockSpec((2 * n, 128), lambda i, q: (i, 0)),
        ],
        out_specs=pl.BlockSpec((2, ni, n, N_DISC), lambda i, q: (i, q, 0, 0)),
        out_shape=jax.ShapeDtypeStruct((b, n, n, N_DISC), jnp.float32),
    )(rs, gathered)
    return out
